# baseline (speedup 1.0000x reference)
import functools

import jax
import jax.numpy as jnp
from jax.experimental import pallas as pl
from jax.experimental.pallas import tpu as pltpu


def _body(x_ref, w_ref, b_ref, out_ref, acc_ref, *, nsteps, s_total):
    i = pl.program_id(0)

    @pl.when(i == 0)
    def _init():
        acc_ref[...] = jnp.zeros_like(acc_ref)

    acc_ref[...] += jnp.sum(x_ref[...], axis=1)

    @pl.when(i == nsteps - 1)
    def _finish():
        pooled = acc_ref[...] * (1.0 / s_total)
        logits = jax.lax.dot_general(
            pooled, w_ref[...],
            dimension_numbers=(((1,), (1,)), ((), ())),
            preferred_element_type=jnp.float32,
        ) + b_ref[...]
        m = jnp.max(logits, axis=-1, keepdims=True)
        e = jnp.exp(logits - m)
        out_ref[...] = e / jnp.sum(e, axis=-1, keepdims=True)


def kernel(x, gate_weight, gate_bias):
    B, S, D = x.shape
    M = gate_weight.shape[0]
    s_blk = 256
    while S % s_blk != 0:
        s_blk //= 2
    nsteps = S // s_blk

    bias2d = gate_bias.reshape(1, M)

    return pl.pallas_call(
        functools.partial(_body, nsteps=nsteps, s_total=S),
        grid=(nsteps,),
        in_specs=[
            pl.BlockSpec((B, s_blk, D), lambda i: (0, i, 0)),
            pl.BlockSpec((M, D), lambda i: (0, 0)),
            pl.BlockSpec((1, M), lambda i: (0, 0)),
        ],
        out_specs=pl.BlockSpec((B, M), lambda i: (0, 0)),
        out_shape=jax.ShapeDtypeStruct((B, M), jnp.float32),
        scratch_shapes=[pltpu.VMEM((B, D), jnp.float32)],
    )(x, gate_weight, bias2d)


# speedup vs baseline: 1.1218x; 1.1218x over previous
import functools

import jax
import jax.numpy as jnp
from jax.experimental import pallas as pl
from jax.experimental.pallas import tpu as pltpu


def _body(x_ref, w_ref, b_ref, out_ref, acc_ref, *, nsteps, s_total):
    b = pl.program_id(0)
    j = pl.program_id(1)

    @pl.when(j == 0)
    def _init():
        acc_ref[...] = jnp.zeros_like(acc_ref)

    acc_ref[...] += jnp.sum(x_ref[...], axis=1)

    @pl.when(j == nsteps - 1)
    def _finish():
        pooled = acc_ref[...] * (1.0 / s_total)
        logits = jax.lax.dot_general(
            pooled, w_ref[...],
            dimension_numbers=(((1,), (1,)), ((), ())),
            preferred_element_type=jnp.float32,
        ) + b_ref[...]
        m = jnp.max(logits, axis=-1, keepdims=True)
        e = jnp.exp(logits - m)
        out_ref[pl.ds(b, 1), :] = e / jnp.sum(e, axis=-1, keepdims=True)


def kernel(x, gate_weight, gate_bias):
    B, S, D = x.shape
    M = gate_weight.shape[0]
    s_blk = 1024
    while S % s_blk != 0:
        s_blk //= 2
    nsteps = S // s_blk

    bias2d = gate_bias.reshape(1, M)

    return pl.pallas_call(
        functools.partial(_body, nsteps=nsteps, s_total=S),
        grid=(B, nsteps),
        in_specs=[
            pl.BlockSpec((1, s_blk, D), lambda b, j: (b, j, 0)),
            pl.BlockSpec((M, D), lambda b, j: (0, 0)),
            pl.BlockSpec((1, M), lambda b, j: (0, 0)),
        ],
        out_specs=pl.BlockSpec((B, M), lambda b, j: (0, 0)),
        out_shape=jax.ShapeDtypeStruct((B, M), jnp.float32),
        scratch_shapes=[pltpu.VMEM((1, D), jnp.float32)],
    )(x, gate_weight, bias2d)
